# baseline (speedup 1.0000x reference)
import jax
import jax.numpy as jnp
from jax.experimental import pallas as pl
from jax.experimental.pallas import tpu as pltpu

_Q = 1024
_KB = 1024
_QC = 8
_SENTINEL = 2.0e17


def _scan_kernel(qt_ref, keys_ref, ids_ref, acc_min, acc_id):
    pid = pl.program_id(0)
    nblk = pl.num_programs(0)

    @pl.when(pid == 0)
    def _init():
        acc_min[...] = jnp.full((1, _Q), jnp.inf, jnp.float32)
        acc_id[...] = jnp.zeros((1, _Q), jnp.int32)

    qx = qt_ref[0:1, :]
    qy = qt_ref[1:2, :]
    kx = keys_ref[:, 0:1]
    ky = keys_ref[:, 1:2]

    ox = qx - kx
    oy = qy - ky
    d2 = ox * ox + oy * oy

    blk_min = jnp.min(d2, axis=0, keepdims=True)
    upd = blk_min < acc_min[...]
    acc_id[...] = jnp.where(upd, pid, acc_id[...])
    acc_min[...] = jnp.where(upd, blk_min, acc_min[...])

    @pl.when(pid == nblk - 1)
    def _finish():
        ids_ref[...] = acc_id[...]


def _winner_kernel(ids_ref, qx_ref, qy_ref, *refs):
    kbs = refs[:_QC]
    out_ref = refs[_QC]
    num_acc = refs[_QC + 1]
    den_acc = refs[_QC + 2]
    pid = pl.program_id(0)
    nstep = pl.num_programs(0)

    @pl.when(pid == 0)
    def _init():
        num_acc[0] = 0.0
        den_acc[0] = 0.0

    num = num_acc[0]
    den = den_acc[0]
    for j in range(_QC):
        q = pid * _QC + j
        kb = kbs[j][0]
        kx = kb[0:2, :]
        ky = kb[2:4, :]
        dx = kb[4:6, :]
        dy = kb[6:8, :]
        qxs = qx_ref[q]
        qys = qy_ref[q]
        ox = qxs - kx
        oy = qys - ky
        d2 = ox * ox + oy * oy
        s = dx * oy - dy * ox
        m = jnp.min(d2)
        sat = jnp.sum(jnp.where(d2 == m, s, 0.0))
        dist = jnp.sqrt(jnp.maximum(m, 1e-12))
        a = jnp.maximum(1.0 + dist * jnp.sign(sat), 0.0)
        num = num + a
        den = den + (a > 0).astype(jnp.float32)
    num_acc[0] = num
    den_acc[0] = den

    @pl.when(pid == nstep - 1)
    def _finish():
        out_ref[...] = (num / (den + 1e-06)).reshape(1, 1)


def kernel(traj, roadgraph_xyz, roadgraph_dir):
    k = roadgraph_xyz.shape[0]
    kpad = ((k + _KB - 1) // _KB) * _KB
    pad = kpad - k
    xyz = jnp.pad(roadgraph_xyz, ((0, pad), (0, 0)), constant_values=_SENTINEL)
    dirs = jnp.pad(roadgraph_dir, ((0, pad), (0, 0)))
    keys = jnp.concatenate([xyz, dirs], axis=1)
    qt = traj.T
    nblk = kpad // _KB

    ids = pl.pallas_call(
        _scan_kernel,
        grid=(nblk,),
        in_specs=[
            pl.BlockSpec((2, _Q), lambda i: (0, 0)),
            pl.BlockSpec((_KB, 4), lambda i: (i, 0)),
        ],
        out_specs=pl.BlockSpec((1, _Q), lambda i: (0, 0)),
        out_shape=jax.ShapeDtypeStruct((1, _Q), jnp.int32),
        scratch_shapes=[
            pltpu.VMEM((1, _Q), jnp.float32),
            pltpu.VMEM((1, _Q), jnp.int32),
        ],
    )(qt, keys)

    blocked = jnp.concatenate(
        [xyz[:, 0].reshape(nblk, 2, 512), xyz[:, 1].reshape(nblk, 2, 512),
         dirs[:, 0].reshape(nblk, 2, 512), dirs[:, 1].reshape(nblk, 2, 512)],
        axis=1)

    ids_flat = ids.reshape(_Q)
    qx = traj[:, 0]
    qy = traj[:, 1]

    nstep = _Q // _QC
    kb_specs = [
        pl.BlockSpec((1, 8, 512),
                     (lambda jj: lambda i, ids_r, qx_r, qy_r:
                      (ids_r[i * _QC + jj], 0, 0))(j))
        for j in range(_QC)
    ]
    loss = pl.pallas_call(
        _winner_kernel,
        grid_spec=pltpu.PrefetchScalarGridSpec(
            num_scalar_prefetch=3,
            grid=(nstep,),
            in_specs=kb_specs,
            out_specs=pl.BlockSpec((1, 1), lambda i, *_: (0, 0)),
            scratch_shapes=[
                pltpu.SMEM((1,), jnp.float32),
                pltpu.SMEM((1,), jnp.float32),
            ],
        ),
        out_shape=jax.ShapeDtypeStruct((1, 1), jnp.float32),
    )(ids_flat, qx, qy, *([blocked] * _QC))
    return loss[0, 0]


# speedup vs baseline: 1.0397x; 1.0397x over previous
import jax
import jax.numpy as jnp
from jax.experimental import pallas as pl
from jax.experimental.pallas import tpu as pltpu

_Q = 1024
_KB = 1024
_SB = 128
_NSUB = _KB // _SB
_QC = 32
_SENTINEL = 2.0e17


def _scan_kernel(qt_ref, keys_ref, ids_ref, acc_min, acc_id):
    pid = pl.program_id(0)
    nblk = pl.num_programs(0)

    @pl.when(pid == 0)
    def _init():
        acc_min[...] = jnp.full((1, _Q), jnp.inf, jnp.float32)
        acc_id[...] = jnp.zeros((1, _Q), jnp.int32)

    qx = qt_ref[0:1, :]
    qy = qt_ref[1:2, :]
    kx = keys_ref[:, 0:1]
    ky = keys_ref[:, 1:2]

    ox = qx - kx
    oy = qy - ky
    d2 = ox * ox + oy * oy

    subs = [jnp.min(d2[i * _SB:(i + 1) * _SB, :], axis=0, keepdims=True)
            for i in range(_NSUB)]
    m8 = jnp.concatenate(subs, axis=0)
    gmin = jnp.min(m8, axis=0, keepdims=True)
    iota = jax.lax.broadcasted_iota(jnp.int32, (_NSUB, _Q), 0)
    sid = jnp.min(jnp.where(m8 == gmin, iota, _NSUB), axis=0,
                  keepdims=True)

    upd = gmin < acc_min[...]
    acc_id[...] = jnp.where(upd, pid * _NSUB + sid, acc_id[...])
    acc_min[...] = jnp.where(upd, gmin, acc_min[...])

    @pl.when(pid == nblk - 1)
    def _finish():
        ids_ref[...] = acc_id[...]


def _winner_kernel(ids_ref, qx_ref, qy_ref, *refs):
    kbs = refs[:_QC]
    out_ref = refs[_QC]
    num_acc = refs[_QC + 1]
    den_acc = refs[_QC + 2]
    pid = pl.program_id(0)
    nstep = pl.num_programs(0)

    @pl.when(pid == 0)
    def _init():
        num_acc[0] = 0.0
        den_acc[0] = 0.0

    num = num_acc[0]
    den = den_acc[0]
    for j in range(_QC):
        q = pid * _QC + j
        kb = kbs[j][0]
        kx = kb[0:1, :]
        ky = kb[1:2, :]
        dx = kb[2:3, :]
        dy = kb[3:4, :]
        qxs = qx_ref[q]
        qys = qy_ref[q]
        ox = qxs - kx
        oy = qys - ky
        d2 = ox * ox + oy * oy
        s = dx * oy - dy * ox
        m = jnp.min(d2)
        sat = jnp.sum(jnp.where(d2 == m, s, 0.0))
        dist = jnp.sqrt(jnp.maximum(m, 1e-12))
        a = jnp.maximum(1.0 + dist * jnp.sign(sat), 0.0)
        num = num + a
        den = den + (a > 0).astype(jnp.float32)
    num_acc[0] = num
    den_acc[0] = den

    @pl.when(pid == nstep - 1)
    def _finish():
        out_ref[...] = (num / (den + 1e-06)).reshape(1, 1)


def kernel(traj, roadgraph_xyz, roadgraph_dir):
    k = roadgraph_xyz.shape[0]
    kpad = ((k + _KB - 1) // _KB) * _KB
    pad = kpad - k
    xyz = jnp.pad(roadgraph_xyz, ((0, pad), (0, 0)), constant_values=_SENTINEL)
    dirs = jnp.pad(roadgraph_dir, ((0, pad), (0, 0)))
    keys = jnp.concatenate([xyz, dirs], axis=1)
    qt = traj.T
    nblk = kpad // _KB
    nsub = kpad // _SB

    ids = pl.pallas_call(
        _scan_kernel,
        grid=(nblk,),
        in_specs=[
            pl.BlockSpec((2, _Q), lambda i: (0, 0)),
            pl.BlockSpec((_KB, 4), lambda i: (i, 0)),
        ],
        out_specs=pl.BlockSpec((1, _Q), lambda i: (0, 0)),
        out_shape=jax.ShapeDtypeStruct((1, _Q), jnp.int32),
        scratch_shapes=[
            pltpu.VMEM((1, _Q), jnp.float32),
            pltpu.VMEM((1, _Q), jnp.int32),
        ],
    )(qt, keys)

    blocked = jnp.concatenate(
        [xyz[:, 0].reshape(nsub, 1, _SB), xyz[:, 1].reshape(nsub, 1, _SB),
         dirs[:, 0].reshape(nsub, 1, _SB), dirs[:, 1].reshape(nsub, 1, _SB)],
        axis=1)

    ids_flat = ids.reshape(_Q)
    qx = traj[:, 0]
    qy = traj[:, 1]

    nstep = _Q // _QC
    kb_specs = [
        pl.BlockSpec((1, 4, _SB),
                     (lambda jj: lambda i, ids_r, qx_r, qy_r:
                      (ids_r[i * _QC + jj], 0, 0))(j))
        for j in range(_QC)
    ]
    loss = pl.pallas_call(
        _winner_kernel,
        grid_spec=pltpu.PrefetchScalarGridSpec(
            num_scalar_prefetch=3,
            grid=(nstep,),
            in_specs=kb_specs,
            out_specs=pl.BlockSpec((1, 1), lambda i, *_: (0, 0)),
            scratch_shapes=[
                pltpu.SMEM((1,), jnp.float32),
                pltpu.SMEM((1,), jnp.float32),
            ],
        ),
        out_shape=jax.ShapeDtypeStruct((1, 1), jnp.float32),
    )(ids_flat, qx, qy, *([blocked] * _QC))
    return loss[0, 0]


# speedup vs baseline: 1.7630x; 1.6958x over previous
import functools

import jax
import jax.numpy as jnp
from jax import lax
from jax.experimental import pallas as pl
from jax.experimental.pallas import tpu as pltpu
from jax.experimental.pallas import tpu_sc as plsc

_Q = 1024
_KB = 1024
_SB = 128
_NSUB = _KB // _SB
_SENTINEL = 2.0e17

_NC = 2
_NS = 16
_NW = _NC * _NS
_QPW = _Q // _NW
_L = 16


def _scan_kernel(qt_ref, keys_ref, ids_ref, acc_min, acc_id):
    pid = pl.program_id(0)
    nblk = pl.num_programs(0)

    @pl.when(pid == 0)
    def _init():
        acc_min[...] = jnp.full((1, _Q), jnp.inf, jnp.float32)
        acc_id[...] = jnp.zeros((1, _Q), jnp.int32)

    qx = qt_ref[0:1, :]
    qy = qt_ref[1:2, :]
    kx = keys_ref[:, 0:1]
    ky = keys_ref[:, 1:2]

    ox = qx - kx
    oy = qy - ky
    d2 = ox * ox + oy * oy

    subs = [jnp.min(d2[i * _SB:(i + 1) * _SB, :], axis=0, keepdims=True)
            for i in range(_NSUB)]
    m8 = jnp.concatenate(subs, axis=0)
    gmin = jnp.min(m8, axis=0, keepdims=True)
    iota = jax.lax.broadcasted_iota(jnp.int32, (_NSUB, _Q), 0)
    sid = jnp.min(jnp.where(m8 == gmin, iota, _NSUB), axis=0,
                  keepdims=True)

    upd = gmin < acc_min[...]
    acc_id[...] = jnp.where(upd, pid * _NSUB + sid, acc_id[...])
    acc_min[...] = jnp.where(upd, gmin, acc_min[...])

    @pl.when(pid == nblk - 1)
    def _finish():
        ids_ref[...] = acc_id[...]


def _sc_winner_kernel(table_hbm, ids_hbm, qx_hbm, qy_hbm, md_hbm, sv_hbm,
                      idx_v, rows_v, qx_v, qy_v, md_v, sv_v, sem):
    wid = lax.axis_index("s") * _NC + lax.axis_index("c")
    base = wid * _QPW
    pltpu.sync_copy(ids_hbm.at[pl.ds(base, _QPW)], idx_v)
    pltpu.sync_copy(qx_hbm.at[pl.ds(base * _L, _QPW * _L)], qx_v)
    pltpu.sync_copy(qy_hbm.at[pl.ds(base * _L, _QPW * _L)], qy_v)
    pltpu.async_copy(table_hbm.at[idx_v], rows_v, sem).wait()
    for t in range(_QPW):
        dmin = jnp.full((_L,), jnp.inf, jnp.float32)
        smin = jnp.zeros((_L,), jnp.float32)
        qxv = qx_v[pl.ds(t * _L, _L)]
        qyv = qy_v[pl.ds(t * _L, _L)]
        for i in range(_SB // _L):
            kxv = rows_v[t, pl.ds(i * _L, _L)]
            kyv = rows_v[t, pl.ds(_SB + i * _L, _L)]
            dxv = rows_v[t, pl.ds(2 * _SB + i * _L, _L)]
            dyv = rows_v[t, pl.ds(3 * _SB + i * _L, _L)]
            ox = qxv - kxv
            oy = qyv - kyv
            d2 = ox * ox + oy * oy
            s = dxv * oy - dyv * ox
            upd = d2 < dmin
            smin = jnp.where(upd, s, smin)
            dmin = jnp.where(upd, d2, dmin)
        md_v[pl.ds(t * _L, _L)] = dmin
        sv_v[pl.ds(t * _L, _L)] = smin
    pltpu.sync_copy(md_v, md_hbm.at[pl.ds(base * _L, _QPW * _L)])
    pltpu.sync_copy(sv_v, sv_hbm.at[pl.ds(base * _L, _QPW * _L)])


def _loss_kernel(md_ref, sv_ref, out_ref):
    md = md_ref[...]
    sv = sv_ref[...]
    m = jnp.min(md, axis=1, keepdims=True)
    sat = jnp.sum(jnp.where(md == m, sv, 0.0), axis=1, keepdims=True)
    dist = jnp.sqrt(jnp.maximum(m, 1e-12))
    a = jnp.maximum(1.0 + dist * jnp.sign(sat), 0.0)
    num = jnp.sum(a)
    den = jnp.sum((a > 0).astype(jnp.float32)) + 1e-06
    out_ref[...] = (num / den).reshape(1, 1)


def kernel(traj, roadgraph_xyz, roadgraph_dir):
    k = roadgraph_xyz.shape[0]
    kpad = ((k + _KB - 1) // _KB) * _KB
    pad = kpad - k
    xyz = jnp.pad(roadgraph_xyz, ((0, pad), (0, 0)), constant_values=_SENTINEL)
    dirs = jnp.pad(roadgraph_dir, ((0, pad), (0, 0)))
    keys = jnp.concatenate([xyz, dirs], axis=1)
    qt = traj.T
    nblk = kpad // _KB
    nsub = kpad // _SB

    ids = pl.pallas_call(
        _scan_kernel,
        grid=(nblk,),
        in_specs=[
            pl.BlockSpec((2, _Q), lambda i: (0, 0)),
            pl.BlockSpec((_KB, 4), lambda i: (i, 0)),
        ],
        out_specs=pl.BlockSpec((1, _Q), lambda i: (0, 0)),
        out_shape=jax.ShapeDtypeStruct((1, _Q), jnp.int32),
        scratch_shapes=[
            pltpu.VMEM((1, _Q), jnp.float32),
            pltpu.VMEM((1, _Q), jnp.int32),
        ],
    )(qt, keys)

    table = jnp.concatenate(
        [xyz[:, 0].reshape(nsub, _SB), xyz[:, 1].reshape(nsub, _SB),
         dirs[:, 0].reshape(nsub, _SB), dirs[:, 1].reshape(nsub, _SB)],
        axis=1)

    mesh = plsc.VectorSubcoreMesh(core_axis_name="c", subcore_axis_name="s")
    sc_phase2 = functools.partial(
        pl.kernel,
        mesh=mesh,
        out_type=[jax.ShapeDtypeStruct((_Q * _L,), jnp.float32),
                  jax.ShapeDtypeStruct((_Q * _L,), jnp.float32)],
        scratch_types=[
            pltpu.VMEM((_QPW,), jnp.int32),
            pltpu.VMEM((_QPW, 4 * _SB), jnp.float32),
            pltpu.VMEM((_QPW * _L,), jnp.float32),
            pltpu.VMEM((_QPW * _L,), jnp.float32),
            pltpu.VMEM((_QPW * _L,), jnp.float32),
            pltpu.VMEM((_QPW * _L,), jnp.float32),
            pltpu.SemaphoreType.DMA,
        ],
    )(_sc_winner_kernel)
    qxb = jnp.broadcast_to(traj[:, 0][:, None], (_Q, _L)).reshape(-1)
    qyb = jnp.broadcast_to(traj[:, 1][:, None], (_Q, _L)).reshape(-1)
    md, sv = sc_phase2(table, ids.reshape(_Q), qxb, qyb)

    loss = pl.pallas_call(
        _loss_kernel,
        out_shape=jax.ShapeDtypeStruct((1, 1), jnp.float32),
    )(md.reshape(_Q, _L), sv.reshape(_Q, _L))
    return loss[0, 0]


# speedup vs baseline: 1.8595x; 1.0547x over previous
import functools

import jax
import jax.numpy as jnp
from jax import lax
from jax.experimental import pallas as pl
from jax.experimental.pallas import tpu as pltpu
from jax.experimental.pallas import tpu_sc as plsc

_Q = 1024
_KB = 2048
_SB = 128
_NSUB = _KB // _SB
_SENTINEL = 2.0e17

_NC = 2
_NS = 16
_NW = _NC * _NS
_QPW = _Q // _NW
_L = 16


def _scan_kernel(qt_ref, keys_ref, ids_ref, acc_min, acc_id):
    pid = pl.program_id(0)
    nblk = pl.num_programs(0)

    @pl.when(pid == 0)
    def _init():
        acc_min[...] = jnp.full((1, _Q), jnp.inf, jnp.float32)
        acc_id[...] = jnp.zeros((1, _Q), jnp.int32)

    qx = qt_ref[0:1, :]
    qy = qt_ref[1:2, :]
    kx = keys_ref[:, 0:1]
    ky = keys_ref[:, 1:2]

    ox = qx - kx
    oy = qy - ky
    d2 = ox * ox + oy * oy

    subs = [jnp.min(d2[i * _SB:(i + 1) * _SB, :], axis=0, keepdims=True)
            for i in range(_NSUB)]
    m8 = jnp.concatenate(subs, axis=0)
    gmin = jnp.min(m8, axis=0, keepdims=True)
    iota = jax.lax.broadcasted_iota(jnp.int32, (_NSUB, _Q), 0)
    sid = jnp.min(jnp.where(m8 == gmin, iota, _NSUB), axis=0,
                  keepdims=True)

    upd = gmin < acc_min[...]
    acc_id[...] = jnp.where(upd, pid * _NSUB + sid, acc_id[...])
    acc_min[...] = jnp.where(upd, gmin, acc_min[...])

    @pl.when(pid == nblk - 1)
    def _finish():
        ids_ref[...] = acc_id[...]


def _sc_winner_kernel(table_hbm, ids_hbm, qx_hbm, qy_hbm, md_hbm, sv_hbm,
                      idx_v, rows_v, qx_v, qy_v, md_v, sv_v, sem):
    wid = lax.axis_index("s") * _NC + lax.axis_index("c")
    base = wid * _QPW
    pltpu.sync_copy(ids_hbm.at[pl.ds(base, _QPW)], idx_v)
    pltpu.sync_copy(qx_hbm.at[pl.ds(base * _L, _QPW * _L)], qx_v)
    pltpu.sync_copy(qy_hbm.at[pl.ds(base * _L, _QPW * _L)], qy_v)
    pltpu.async_copy(table_hbm.at[idx_v], rows_v, sem).wait()
    for t in range(_QPW):
        dmin = jnp.full((_L,), jnp.inf, jnp.float32)
        smin = jnp.zeros((_L,), jnp.float32)
        qxv = qx_v[pl.ds(t * _L, _L)]
        qyv = qy_v[pl.ds(t * _L, _L)]
        for i in range(_SB // _L):
            kxv = rows_v[t, pl.ds(i * _L, _L)]
            kyv = rows_v[t, pl.ds(_SB + i * _L, _L)]
            dxv = rows_v[t, pl.ds(2 * _SB + i * _L, _L)]
            dyv = rows_v[t, pl.ds(3 * _SB + i * _L, _L)]
            ox = qxv - kxv
            oy = qyv - kyv
            d2 = ox * ox + oy * oy
            s = dxv * oy - dyv * ox
            upd = d2 < dmin
            smin = jnp.where(upd, s, smin)
            dmin = jnp.where(upd, d2, dmin)
        md_v[pl.ds(t * _L, _L)] = dmin
        sv_v[pl.ds(t * _L, _L)] = smin
    pltpu.sync_copy(md_v, md_hbm.at[pl.ds(base * _L, _QPW * _L)])
    pltpu.sync_copy(sv_v, sv_hbm.at[pl.ds(base * _L, _QPW * _L)])


def _loss_kernel(md_ref, sv_ref, out_ref):
    md = md_ref[...]
    sv = sv_ref[...]
    m = jnp.min(md, axis=1, keepdims=True)
    sat = jnp.sum(jnp.where(md == m, sv, 0.0), axis=1, keepdims=True)
    dist = jnp.sqrt(jnp.maximum(m, 1e-12))
    a = jnp.maximum(1.0 + dist * jnp.sign(sat), 0.0)
    num = jnp.sum(a)
    den = jnp.sum((a > 0).astype(jnp.float32)) + 1e-06
    out_ref[...] = (num / den).reshape(1, 1)


def kernel(traj, roadgraph_xyz, roadgraph_dir):
    k = roadgraph_xyz.shape[0]
    kpad = ((k + _KB - 1) // _KB) * _KB
    pad = kpad - k
    xyz = jnp.pad(roadgraph_xyz, ((0, pad), (0, 0)), constant_values=_SENTINEL)
    dirs = jnp.pad(roadgraph_dir, ((0, pad), (0, 0)))
    keys = jnp.concatenate([xyz, dirs], axis=1)
    qt = traj.T
    nblk = kpad // _KB
    nsub = kpad // _SB

    ids = pl.pallas_call(
        _scan_kernel,
        grid=(nblk,),
        in_specs=[
            pl.BlockSpec((2, _Q), lambda i: (0, 0)),
            pl.BlockSpec((_KB, 4), lambda i: (i, 0)),
        ],
        out_specs=pl.BlockSpec((1, _Q), lambda i: (0, 0)),
        out_shape=jax.ShapeDtypeStruct((1, _Q), jnp.int32),
        scratch_shapes=[
            pltpu.VMEM((1, _Q), jnp.float32),
            pltpu.VMEM((1, _Q), jnp.int32),
        ],
    )(qt, keys)

    table = jnp.concatenate(
        [xyz[:, 0].reshape(nsub, _SB), xyz[:, 1].reshape(nsub, _SB),
         dirs[:, 0].reshape(nsub, _SB), dirs[:, 1].reshape(nsub, _SB)],
        axis=1)

    mesh = plsc.VectorSubcoreMesh(core_axis_name="c", subcore_axis_name="s")
    sc_phase2 = functools.partial(
        pl.kernel,
        mesh=mesh,
        out_type=[jax.ShapeDtypeStruct((_Q * _L,), jnp.float32),
                  jax.ShapeDtypeStruct((_Q * _L,), jnp.float32)],
        scratch_types=[
            pltpu.VMEM((_QPW,), jnp.int32),
            pltpu.VMEM((_QPW, 4 * _SB), jnp.float32),
            pltpu.VMEM((_QPW * _L,), jnp.float32),
            pltpu.VMEM((_QPW * _L,), jnp.float32),
            pltpu.VMEM((_QPW * _L,), jnp.float32),
            pltpu.VMEM((_QPW * _L,), jnp.float32),
            pltpu.SemaphoreType.DMA,
        ],
    )(_sc_winner_kernel)
    qxb = jnp.broadcast_to(traj[:, 0][:, None], (_Q, _L)).reshape(-1)
    qyb = jnp.broadcast_to(traj[:, 1][:, None], (_Q, _L)).reshape(-1)
    md, sv = sc_phase2(table, ids.reshape(_Q), qxb, qyb)

    loss = pl.pallas_call(
        _loss_kernel,
        out_shape=jax.ShapeDtypeStruct((1, 1), jnp.float32),
    )(md.reshape(_Q, _L), sv.reshape(_Q, _L))
    return loss[0, 0]


# speedup vs baseline: 1.8772x; 1.0095x over previous
import functools

import jax
import jax.numpy as jnp
from jax import lax
from jax.experimental import pallas as pl
from jax.experimental.pallas import tpu as pltpu
from jax.experimental.pallas import tpu_sc as plsc

_Q = 1024
_KB = 4096
_SB = 128
_NSUB = _KB // _SB
_SENTINEL = 2.0e17

_NC = 2
_NS = 16
_NW = _NC * _NS
_QPW = _Q // _NW
_L = 16


def _scan_kernel(qt_ref, keys_ref, ids_ref, acc_min, acc_id):
    pid = pl.program_id(0)
    nblk = pl.num_programs(0)

    @pl.when(pid == 0)
    def _init():
        acc_min[...] = jnp.full((1, _Q), jnp.inf, jnp.float32)
        acc_id[...] = jnp.zeros((1, _Q), jnp.int32)

    qx = qt_ref[0:1, :]
    qy = qt_ref[1:2, :]
    kx = keys_ref[:, 0:1]
    ky = keys_ref[:, 1:2]

    ox = qx - kx
    oy = qy - ky
    d2 = ox * ox + oy * oy

    subs = [jnp.min(d2[i * _SB:(i + 1) * _SB, :], axis=0, keepdims=True)
            for i in range(_NSUB)]
    m8 = jnp.concatenate(subs, axis=0)
    gmin = jnp.min(m8, axis=0, keepdims=True)
    iota = jax.lax.broadcasted_iota(jnp.int32, (_NSUB, _Q), 0)
    sid = jnp.min(jnp.where(m8 == gmin, iota, _NSUB), axis=0,
                  keepdims=True)

    upd = gmin < acc_min[...]
    acc_id[...] = jnp.where(upd, pid * _NSUB + sid, acc_id[...])
    acc_min[...] = jnp.where(upd, gmin, acc_min[...])

    @pl.when(pid == nblk - 1)
    def _finish():
        ids_ref[...] = acc_id[...]


def _sc_winner_kernel(table_hbm, ids_hbm, qx_hbm, qy_hbm, md_hbm, sv_hbm,
                      idx_v, rows_v, qx_v, qy_v, md_v, sv_v, sem):
    wid = lax.axis_index("s") * _NC + lax.axis_index("c")
    base = wid * _QPW
    pltpu.sync_copy(ids_hbm.at[pl.ds(base, _QPW)], idx_v)
    pltpu.sync_copy(qx_hbm.at[pl.ds(base * _L, _QPW * _L)], qx_v)
    pltpu.sync_copy(qy_hbm.at[pl.ds(base * _L, _QPW * _L)], qy_v)
    pltpu.async_copy(table_hbm.at[idx_v], rows_v, sem).wait()
    for t in range(_QPW):
        dmin = jnp.full((_L,), jnp.inf, jnp.float32)
        smin = jnp.zeros((_L,), jnp.float32)
        qxv = qx_v[pl.ds(t * _L, _L)]
        qyv = qy_v[pl.ds(t * _L, _L)]
        for i in range(_SB // _L):
            kxv = rows_v[t, pl.ds(i * _L, _L)]
            kyv = rows_v[t, pl.ds(_SB + i * _L, _L)]
            dxv = rows_v[t, pl.ds(2 * _SB + i * _L, _L)]
            dyv = rows_v[t, pl.ds(3 * _SB + i * _L, _L)]
            ox = qxv - kxv
            oy = qyv - kyv
            d2 = ox * ox + oy * oy
            s = dxv * oy - dyv * ox
            upd = d2 < dmin
            smin = jnp.where(upd, s, smin)
            dmin = jnp.where(upd, d2, dmin)
        md_v[pl.ds(t * _L, _L)] = dmin
        sv_v[pl.ds(t * _L, _L)] = smin
    pltpu.sync_copy(md_v, md_hbm.at[pl.ds(base * _L, _QPW * _L)])
    pltpu.sync_copy(sv_v, sv_hbm.at[pl.ds(base * _L, _QPW * _L)])


def _loss_kernel(md_ref, sv_ref, out_ref):
    md = md_ref[...]
    sv = sv_ref[...]
    m = jnp.min(md, axis=1, keepdims=True)
    sat = jnp.sum(jnp.where(md == m, sv, 0.0), axis=1, keepdims=True)
    dist = jnp.sqrt(jnp.maximum(m, 1e-12))
    a = jnp.maximum(1.0 + dist * jnp.sign(sat), 0.0)
    num = jnp.sum(a)
    den = jnp.sum((a > 0).astype(jnp.float32)) + 1e-06
    out_ref[...] = (num / den).reshape(1, 1)


def kernel(traj, roadgraph_xyz, roadgraph_dir):
    k = roadgraph_xyz.shape[0]
    kpad = ((k + _KB - 1) // _KB) * _KB
    pad = kpad - k
    xyz = jnp.pad(roadgraph_xyz, ((0, pad), (0, 0)), constant_values=_SENTINEL)
    dirs = jnp.pad(roadgraph_dir, ((0, pad), (0, 0)))
    keys = jnp.concatenate([xyz, dirs], axis=1)
    qt = traj.T
    nblk = kpad // _KB
    nsub = kpad // _SB

    ids = pl.pallas_call(
        _scan_kernel,
        grid=(nblk,),
        in_specs=[
            pl.BlockSpec((2, _Q), lambda i: (0, 0)),
            pl.BlockSpec((_KB, 4), lambda i: (i, 0)),
        ],
        out_specs=pl.BlockSpec((1, _Q), lambda i: (0, 0)),
        out_shape=jax.ShapeDtypeStruct((1, _Q), jnp.int32),
        scratch_shapes=[
            pltpu.VMEM((1, _Q), jnp.float32),
            pltpu.VMEM((1, _Q), jnp.int32),
        ],
    )(qt, keys)

    table = jnp.concatenate(
        [xyz[:, 0].reshape(nsub, _SB), xyz[:, 1].reshape(nsub, _SB),
         dirs[:, 0].reshape(nsub, _SB), dirs[:, 1].reshape(nsub, _SB)],
        axis=1)

    mesh = plsc.VectorSubcoreMesh(core_axis_name="c", subcore_axis_name="s")
    sc_phase2 = functools.partial(
        pl.kernel,
        mesh=mesh,
        out_type=[jax.ShapeDtypeStruct((_Q * _L,), jnp.float32),
                  jax.ShapeDtypeStruct((_Q * _L,), jnp.float32)],
        scratch_types=[
            pltpu.VMEM((_QPW,), jnp.int32),
            pltpu.VMEM((_QPW, 4 * _SB), jnp.float32),
            pltpu.VMEM((_QPW * _L,), jnp.float32),
            pltpu.VMEM((_QPW * _L,), jnp.float32),
            pltpu.VMEM((_QPW * _L,), jnp.float32),
            pltpu.VMEM((_QPW * _L,), jnp.float32),
            pltpu.SemaphoreType.DMA,
        ],
    )(_sc_winner_kernel)
    qxb = jnp.broadcast_to(traj[:, 0][:, None], (_Q, _L)).reshape(-1)
    qyb = jnp.broadcast_to(traj[:, 1][:, None], (_Q, _L)).reshape(-1)
    md, sv = sc_phase2(table, ids.reshape(_Q), qxb, qyb)

    loss = pl.pallas_call(
        _loss_kernel,
        out_shape=jax.ShapeDtypeStruct((1, 1), jnp.float32),
    )(md.reshape(_Q, _L), sv.reshape(_Q, _L))
    return loss[0, 0]
